# baseline (speedup 1.0000x reference)
import functools

import jax
import jax.numpy as jnp
from jax import lax
from jax.experimental import pallas as pl
from jax.experimental.pallas import tpu as pltpu
from jax.experimental.pallas import tpu_sc as plsc

_CH = 1024
_SC_ROWS = 6144
_CHUNK = 32
_CHUNKS_PER_W = _SC_ROWS // (32 * _CHUNK)

_mesh = plsc.VectorSubcoreMesh(core_axis_name="c", subcore_axis_name="s")


@functools.partial(
    pl.kernel,
    out_type=jax.ShapeDtypeStruct((_SC_ROWS, _CH), jnp.float32),
    mesh=_mesh,
    scratch_types=[
        pltpu.VMEM((2, _CHUNK, _CH), jnp.float32),
        pltpu.SemaphoreType.DMA,
        pltpu.SemaphoreType.DMA,
    ],
)
def _sc_copy(x_hbm, o_hbm, buf, sem_in, sem_out):
    wid = lax.axis_index("s") * 2 + lax.axis_index("c")
    base = wid * _CHUNKS_PER_W * _CHUNK

    def src_row(j):
        return lax.rem(base + j * _CHUNK, 4096)

    n = _CHUNKS_PER_W
    in_copies = [None] * n
    out_copies = [None] * n
    in_copies[0] = pltpu.async_copy(
        x_hbm.at[pl.ds(src_row(0), _CHUNK)], buf.at[0], sem_in)
    for j in range(n):
        slot = j % 2
        if j + 1 < n:
            if j - 1 >= 0:
                out_copies[j - 1].wait()
            in_copies[j + 1] = pltpu.async_copy(
                x_hbm.at[pl.ds(src_row(j + 1), _CHUNK)], buf.at[1 - slot],
                sem_in)
        in_copies[j].wait()
        out_copies[j] = pltpu.async_copy(
            buf.at[slot], o_hbm.at[pl.ds(base + j * _CHUNK, _CHUNK)], sem_out)
    out_copies[n - 2].wait()
    out_copies[n - 1].wait()


def _zero_kernel(o_ref):
    o_ref[...] = jnp.zeros_like(o_ref)


def kernel(pos_embs, batch_size, index_dim):
    del batch_size, index_dim
    big = pl.pallas_call(
        _zero_kernel,
        grid=(8,),
        out_specs=pl.BlockSpec((4, 512, 1024), lambda i: (0, i, 0)),
        out_shape=jax.ShapeDtypeStruct((4, 4096, 1024), jnp.float32),
    )()
    small = _sc_copy(pos_embs)
    return big, small


# speedup vs baseline: 1.7560x; 1.7560x over previous
import jax
import jax.numpy as jnp
from jax.experimental import pallas as pl

_BATCH = 4
_ROWS = 4096
_CH = 1024
_BLK = 1024


def _copy_kernel(x_ref, o_ref):
    o_ref[...] = x_ref[...]


def kernel(pos_embs, batch_size, index_dim):
    del batch_size, index_dim
    nblk = _ROWS // _BLK
    flat = pl.pallas_call(
        _copy_kernel,
        grid=(nblk, _BATCH),
        in_specs=[pl.BlockSpec((_BLK, _CH), lambda i, b: (i, 0))],
        out_specs=pl.BlockSpec((_BLK, _CH), lambda i, b: (b * 4 + i, 0)),
        out_shape=jax.ShapeDtypeStruct((_BATCH * _ROWS, _CH), jnp.float32),
    )(pos_embs)
    return flat.reshape(_BATCH, _ROWS, _CH)


# speedup vs baseline: 1.9069x; 1.0859x over previous
import jax
import jax.numpy as jnp
from jax.experimental import pallas as pl

_BATCH = 4
_ROWS = 4096
_CH = 1024
_BLK = 256


def _bcast_kernel(x_ref, o_ref):
    o_ref[...] = jnp.broadcast_to(x_ref[...][None], o_ref.shape)


def kernel(pos_embs, batch_size, index_dim):
    del batch_size, index_dim
    nblk = _ROWS // _BLK
    return pl.pallas_call(
        _bcast_kernel,
        grid=(nblk,),
        in_specs=[pl.BlockSpec((_BLK, _CH), lambda i: (i, 0))],
        out_specs=pl.BlockSpec((_BATCH, _BLK, _CH), lambda i: (0, i, 0)),
        out_shape=jax.ShapeDtypeStruct((_BATCH, _ROWS, _CH), jnp.float32),
    )(pos_embs)


# speedup vs baseline: 2.1190x; 1.1112x over previous
import jax
import jax.numpy as jnp
from jax.experimental import pallas as pl

_BATCH = 4
_ROWS = 4096
_CH = 1024
_BLK = 1024


def _bcast_kernel(x_ref, o_ref):
    o_ref[...] = jnp.broadcast_to(x_ref[...][None], o_ref.shape)


def kernel(pos_embs, batch_size, index_dim):
    del batch_size, index_dim
    nblk = _ROWS // _BLK
    return pl.pallas_call(
        _bcast_kernel,
        grid=(nblk,),
        in_specs=[pl.BlockSpec((_BLK, _CH), lambda i: (i, 0))],
        out_specs=pl.BlockSpec((_BATCH, _BLK, _CH), lambda i: (0, i, 0)),
        out_shape=jax.ShapeDtypeStruct((_BATCH, _ROWS, _CH), jnp.float32),
    )(pos_embs)
